# baseline (speedup 1.0000x reference)
import functools

import jax
import jax.numpy as jnp
from jax import lax
from jax.experimental import pallas as pl
from jax.experimental.pallas import tpu as pltpu
from jax.experimental.pallas import tpu_sc as plsc

BLANK_TOKEN_ID = 0
NC = 2
NS = 16
L = 16

BATCH = 4
SEQ = 8192
D = 1024
TOKENS = BATCH * SEQ
NW = NC * NS
CHUNK = TOKENS // NW
CHUNKS_PER_ROW = SEQ // CHUNK
C = 16
NBLK = CHUNK // C

BUFROWS = 640
SEGSTRIDE = BUFROWS - 24
NSEG = -(-SEQ // SEGSTRIDE)
ROWS_PER_TILE = BUFROWS // NS
SEG_BASE = [min(SEGSTRIDE * s, SEQ - BUFROWS) for s in range(NSEG)]


def _body(x_hbm, w_hbm, out_hbm, xrow, idx, buf0, buf1,
          segA, segB, tsem, ssem0, ssem1):
    cid = lax.axis_index("c")
    sid = lax.axis_index("s")
    wid = cid * NS + sid
    b = wid // CHUNKS_PER_ROW
    k = wid % CHUNKS_PER_ROW

    pltpu.sync_copy(x_hbm.at[pl.ds(b * SEQ, SEQ)], xrow)

    def count_body(i, acc):
        v = xrow[pl.ds(i * L, L)]
        return acc + jnp.where(v == BLANK_TOKEN_ID, jnp.int32(1), jnp.int32(0))

    acc = lax.fori_loop(
        0, k * (CHUNK // L), count_body, jnp.zeros((L,), jnp.int32)
    )
    carry0 = jnp.sum(acc)

    base0 = k * CHUNK
    lane = lax.iota(jnp.int32, L)

    def pos_body(i, carry):
        v = xrow[pl.ds(base0 + i * L, L)]
        isb = jnp.where(v == BLANK_TOKEN_ID, jnp.int32(1), jnp.int32(0))
        cs = plsc.cumsum(isb)
        posv = (base0 + i * L + lane) - (cs + carry)
        idx[pl.ds(i * L, L)] = jnp.maximum(posv, 0)
        return carry + jnp.sum(isb)

    lax.fori_loop(0, CHUNK // L, pos_body, carry0)

    blk_iota = lane * C
    bf = [plsc.load_gather(idx, [blk_iota + q * L * C]) for q in range(NBLK // L)]

    one = jnp.int32(1)
    zero = jnp.int32(0)

    def blocks_below(t):
        counts = [jnp.sum(jnp.where(v < t, one, zero)) for v in bf]
        total = counts[0]
        for cnt in counts[1:]:
            total = total + cnt
        return total

    lo = [jnp.int32(0)] + [
        blocks_below(jnp.int32(SEGSTRIDE * s)) for s in range(1, NSEG)
    ] + [jnp.int32(NBLK)]

    tbase = wid * CHUNK

    def stage_start(s, seg):
        return pltpu.async_copy(
            w_hbm.at[pl.ds((SEG_BASE[s] + sid * ROWS_PER_TILE) * D, ROWS_PER_TILE * D)],
            seg.at[pl.ds(sid * ROWS_PER_TILE * D, ROWS_PER_TILE * D)],
            tsem,
        )

    def stage_wait(s, seg):
        pltpu.make_async_copy(
            w_hbm.at[pl.ds((SEG_BASE[s] + sid * ROWS_PER_TILE) * D, ROWS_PER_TILE * D)],
            seg.at[pl.ds(sid * ROWS_PER_TILE * D, ROWS_PER_TILE * D)],
            tsem,
        ).wait()

    def out_slice(j):
        return out_hbm.at[pl.ds((tbase + j * C) * D, C * D)]

    def scatter_wait(j, seg, sem):
        pltpu.make_async_copy(seg.at[pl.ds(0, C * D)], out_slice(j), sem).wait()

    def do_block(j, seg, seg_base, lo_s, buf, sem):
        rel = idx[pl.ds(j * C, L)] - seg_base
        rel0 = jnp.min(rel)
        contiguous = jnp.sum(jnp.where(rel != rel0 + lane, one, zero)) == 0

        @pl.when(j >= lo_s + 2)
        def _free():
            scatter_wait(j - 2, seg, sem)

        @pl.when(contiguous)
        def _direct():
            pltpu.async_copy(seg.at[pl.ds(rel0 * D, C * D)], out_slice(j), sem)

        @pl.when(jnp.logical_not(contiguous))
        def _fixup():
            def fix(i, _):
                rel_i = jnp.sum(jnp.where(lane == i, rel, zero))
                pltpu.sync_copy(seg.at[pl.ds(rel_i * D, D)], buf.at[pl.ds(i * D, D)])
                return 0

            lax.fori_loop(0, C, fix, 0)
            pltpu.async_copy(buf, out_slice(j), sem)

    def drain(j, seg):
        @pl.when(j % 2 == 0)
        def _d0():
            scatter_wait(j, seg, ssem0)

        @pl.when(j % 2 == 1)
        def _d1():
            scatter_wait(j, seg, ssem1)

    def process(s, seg):
        seg_base = jnp.int32(SEG_BASE[s])
        lo_s, hi_s = lo[s], lo[s + 1]

        def blk(j, _):
            @pl.when(j % 2 == 0)
            def _even():
                do_block(j, seg, seg_base, lo_s, buf0, ssem0)

            @pl.when(j % 2 == 1)
            def _odd():
                do_block(j, seg, seg_base, lo_s, buf1, ssem1)

            return 0

        lax.fori_loop(lo_s, hi_s, blk, 0)

        cnt = hi_s - lo_s

        @pl.when(cnt >= 1)
        def _last():
            drain(hi_s - 1, seg)

        @pl.when(cnt >= 2)
        def _penult():
            drain(hi_s - 2, seg)

    stage_start(0, segA)
    stage_wait(0, segA)
    plsc.subcore_barrier()
    for s in range(NSEG):
        cur, nxt = (segA, segB) if s % 2 == 0 else (segB, segA)
        if s + 1 < NSEG:
            stage_start(s + 1, nxt)
        process(s, cur)
        if s + 1 < NSEG:
            stage_wait(s + 1, nxt)
        plsc.subcore_barrier()


@jax.jit
def kernel(x, W):
    x_flat = x.reshape(TOKENS).astype(jnp.int32)
    W = W.astype(jnp.float32)

    mesh = plsc.VectorSubcoreMesh(
        core_axis_name="c", subcore_axis_name="s", num_cores=NC, num_subcores=NS
    )
    run = pl.kernel(
        _body,
        out_type=jax.ShapeDtypeStruct((TOKENS * D,), jnp.float32),
        mesh=mesh,
        scratch_types=[
            pltpu.VMEM((SEQ,), jnp.int32),
            pltpu.VMEM((CHUNK,), jnp.int32),
            pltpu.VMEM((C * D,), jnp.float32),
            pltpu.VMEM((C * D,), jnp.float32),
            pltpu.VMEM_SHARED((BUFROWS * D,), jnp.float32),
            pltpu.VMEM_SHARED((BUFROWS * D,), jnp.float32),
            pltpu.SemaphoreType.DMA,
            pltpu.SemaphoreType.DMA,
            pltpu.SemaphoreType.DMA,
        ],
        compiler_params=pltpu.CompilerParams(needs_layout_passes=False),
    )
    out = run(x_flat, W.reshape(SEQ * D))
    return out.reshape(BATCH, SEQ, D)


# speedup vs baseline: 3.1489x; 3.1489x over previous
import functools

import jax
import jax.numpy as jnp
from jax import lax
from jax.experimental import pallas as pl
from jax.experimental.pallas import tpu as pltpu
from jax.experimental.pallas import tpu_sc as plsc

BLANK_TOKEN_ID = 0
NC = 2
NS = 16
L = 16

BATCH = 4
SEQ = 8192
D = 1024
TOKENS = BATCH * SEQ
NW = NC * NS
CHUNK = TOKENS // NW
CHUNKS_PER_ROW = SEQ // CHUNK
C = 32
NSTEP = CHUNK // C
NG = NSTEP // 2


def _body(x_hbm, w_hbm, out_hbm, xrow, idx, rows0, rows1, gsem0, gsem1):
    c = lax.axis_index("c")
    s = lax.axis_index("s")
    wid = c * NS + s
    b = wid // CHUNKS_PER_ROW
    k = wid % CHUNKS_PER_ROW

    pltpu.sync_copy(x_hbm.at[pl.ds(b * SEQ, SEQ)], xrow)

    def count_body(i, acc):
        v = xrow[pl.ds(i * L, L)]
        return acc + jnp.where(v == BLANK_TOKEN_ID, jnp.int32(1), jnp.int32(0))

    acc = lax.fori_loop(0, k * (CHUNK // L), count_body, jnp.zeros((L,), jnp.int32))
    carry0 = jnp.sum(acc)

    base0 = k * CHUNK

    def pos_body(i, carry):
        v = xrow[pl.ds(base0 + i * L, L)]
        isb = jnp.where(v == BLANK_TOKEN_ID, jnp.int32(1), jnp.int32(0))
        cs = plsc.cumsum(isb)
        posv = (base0 + i * L + lax.iota(jnp.int32, L)) - (cs + carry)
        idx[pl.ds(i * L, L)] = jnp.maximum(posv, 0)
        return carry + jnp.sum(isb)

    lax.fori_loop(0, CHUNK // L, pos_body, carry0)

    tbase = wid * CHUNK

    def gather_start(step, buf, sem):
        return pltpu.async_copy(w_hbm.at[idx.at[pl.ds(step * C, C)]], buf, sem)

    def gather_wait(step, buf, sem):
        pltpu.make_async_copy(w_hbm.at[idx.at[pl.ds(step * C, C)]], buf, sem).wait()

    def scatter(step, buf):
        pltpu.sync_copy(buf, out_hbm.at[pl.ds(tbase + step * C, C)])

    gather_start(0, rows0, gsem0)

    def grp(g, _):
        a = 2 * g
        gather_start(a + 1, rows1, gsem1)
        gather_wait(a, rows0, gsem0)
        scatter(a, rows0)

        @pl.when(g < NG - 1)
        def _prefetch():
            gather_start(a + 2, rows0, gsem0)

        gather_wait(a + 1, rows1, gsem1)
        scatter(a + 1, rows1)
        return 0

    lax.fori_loop(0, NG, grp, 0)


@jax.jit
def kernel(x, W):
    x_flat = x.reshape(TOKENS).astype(jnp.int32)
    W = W.astype(jnp.float32)

    mesh = plsc.VectorSubcoreMesh(
        core_axis_name="c", subcore_axis_name="s", num_cores=NC, num_subcores=NS
    )
    run = pl.kernel(
        _body,
        out_type=jax.ShapeDtypeStruct((TOKENS, D), jnp.float32),
        mesh=mesh,
        scratch_types=[
            pltpu.VMEM((SEQ,), jnp.int32),
            pltpu.VMEM((CHUNK,), jnp.int32),
            pltpu.VMEM((C, D), jnp.float32),
            pltpu.VMEM((C, D), jnp.float32),
            pltpu.SemaphoreType.DMA,
            pltpu.SemaphoreType.DMA,
        ],
        compiler_params=pltpu.CompilerParams(needs_layout_passes=False),
    )
    out = run(x_flat, W)
    return out.reshape(BATCH, SEQ, D)


# speedup vs baseline: 3.1653x; 1.0052x over previous
import jax
import jax.numpy as jnp
from jax import lax
from jax.experimental import pallas as pl
from jax.experimental.pallas import tpu as pltpu
from jax.experimental.pallas import tpu_sc as plsc

BLANK_TOKEN_ID = 0
NC = 2
NS = 16
L = 16

BATCH = 4
SEQ = 8192
D = 1024
TOKENS = BATCH * SEQ
NW = NC * NS
CHUNK = TOKENS // NW
CHUNKS_PER_ROW = SEQ // CHUNK
C = 32
NSTEP = CHUNK // C
NG = NSTEP // 2


def _body(x_hbm, w_hbm, out_hbm, xrow, idx, rows0, rows1, gsem0, gsem1):
    c = lax.axis_index("c")
    s = lax.axis_index("s")
    wid = c * NS + s
    b = wid // CHUNKS_PER_ROW
    k = wid % CHUNKS_PER_ROW

    pltpu.sync_copy(x_hbm.at[pl.ds(b * SEQ, SEQ)], xrow)

    def count_body(i, acc):
        v = xrow[pl.ds(i * L, L)]
        return acc + jnp.where(v == BLANK_TOKEN_ID, jnp.int32(1), jnp.int32(0))

    acc = lax.fori_loop(0, k * (CHUNK // L), count_body, jnp.zeros((L,), jnp.int32))
    carry0 = jnp.sum(acc)

    base0 = k * CHUNK

    def pos_body(i, carry):
        v = xrow[pl.ds(base0 + i * L, L)]
        isb = jnp.where(v == BLANK_TOKEN_ID, jnp.int32(1), jnp.int32(0))
        cs = plsc.cumsum(isb)
        posv = (base0 + i * L + lax.iota(jnp.int32, L)) - (cs + carry)
        idx[pl.ds(i * L, L)] = jnp.maximum(posv, 0)
        return carry + jnp.sum(isb)

    lax.fori_loop(0, CHUNK // L, pos_body, carry0)

    tbase = wid * CHUNK

    def gather_start(step, buf, sem):
        return pltpu.async_copy(w_hbm.at[idx.at[pl.ds(step * C, C)]], buf, sem)

    def gather_wait(step, buf, sem):
        pltpu.make_async_copy(w_hbm.at[idx.at[pl.ds(step * C, C)]], buf, sem).wait()

    def scatter(step, buf):
        pltpu.sync_copy(buf, out_hbm.at[pl.ds(tbase + step * C, C)])

    gather_start(0, rows0, gsem0)

    def grp(g, _):
        a = 2 * g
        gather_start(a + 1, rows1, gsem1)
        gather_wait(a, rows0, gsem0)
        scatter(a, rows0)

        @pl.when(g < NG - 1)
        def _prefetch():
            gather_start(a + 2, rows0, gsem0)

        gather_wait(a + 1, rows1, gsem1)
        scatter(a + 1, rows1)
        return 0

    lax.fori_loop(0, NG, grp, 0)


@jax.jit
def kernel(x, W):
    x_flat = x.reshape(TOKENS).astype(jnp.int32)
    W = W.astype(jnp.float32)

    mesh = plsc.VectorSubcoreMesh(
        core_axis_name="c", subcore_axis_name="s", num_cores=NC, num_subcores=NS
    )
    run = pl.kernel(
        _body,
        out_type=jax.ShapeDtypeStruct((TOKENS, D), jnp.float32),
        mesh=mesh,
        scratch_types=[
            pltpu.VMEM((SEQ,), jnp.int32),
            pltpu.VMEM((CHUNK,), jnp.int32),
            pltpu.VMEM((C, D), jnp.float32),
            pltpu.VMEM((C, D), jnp.float32),
            pltpu.SemaphoreType.DMA,
            pltpu.SemaphoreType.DMA,
        ],
        compiler_params=pltpu.CompilerParams(needs_layout_passes=False),
    )
    out = run(x_flat, W)
    return out.reshape(BATCH, SEQ, D)
